# baseline (speedup 1.0000x reference)
import jax
import jax.numpy as jnp
from jax.experimental import pallas as pl
from jax.experimental.pallas import tpu as pltpu


def _node_kernel(Tf, Tt_t, adj_t, X, Z, w1, b1, p1, p2, d2_out, acc, hw):
    j = pl.program_id(0)
    nj = pl.num_programs(0)
    BJ = adj_t.shape[1]

    @pl.when(j == 0)
    def _init():
        hw[...] = jnp.dot(X[...], w1[...], preferred_element_type=jnp.float32)
        acc[...] = jnp.zeros_like(acc)

    d1 = jax.lax.dot_general(Z[...], p1[...], (((1,), (1,)), ((), ())),
                             preferred_element_type=jnp.float32)
    W = Tt_t[...] * d1
    mult = jnp.dot(Tf[...], W, preferred_element_type=jnp.float32)

    ii = jax.lax.broadcasted_iota(jnp.int32, mult.shape, 0)
    jj = jax.lax.broadcasted_iota(jnp.int32, mult.shape, 1) + j * BJ
    A = jnp.where(ii == jj, adj_t[...], mult * adj_t[...])
    inv = 1.0 / (jnp.sum(A, axis=0) + 1e-10)
    V = hw[pl.ds(j * BJ, BJ), :] * inv[:, None]
    acc[...] += jnp.dot(A, V, preferred_element_type=jnp.float32)

    @pl.when(j == nj - 1)
    def _fin():
        Xv = jnp.maximum(acc[...] + b1[...], 0.0)
        d2_out[...] = jax.lax.dot_general(
            Xv, p2[...], (((1,), (1,)), ((), ())),
            preferred_element_type=jnp.float32)


def _edge_kernel(Tt, Tcol_t, adj_t, Z, w2, b2, d2, o_ref, acc):
    j = pl.program_id(0)
    nj = pl.num_programs(0)
    BJ = adj_t.shape[1]

    @pl.when(j == 0)
    def _init():
        acc[...] = jnp.zeros_like(acc)

    W = Tcol_t[...] * d2[...]
    mult = jnp.dot(Tt[...], W, preferred_element_type=jnp.float32)

    ii = jax.lax.broadcasted_iota(jnp.int32, mult.shape, 0)
    jj = jax.lax.broadcasted_iota(jnp.int32, mult.shape, 1) + j * BJ
    A = jnp.where(ii == jj, adj_t[...], mult * adj_t[...])
    inv = 1.0 / (jnp.sum(A, axis=0) + 1e-10)

    Ze_t = jnp.maximum(Z[pl.ds(j * BJ, BJ), :], 0.0)
    HW2_t = jnp.dot(Ze_t, w2[...], preferred_element_type=jnp.float32)
    V = HW2_t * inv[:, None]
    acc[...] += jnp.dot(A, V, preferred_element_type=jnp.float32)

    @pl.when(j == nj - 1)
    def _fin():
        out2 = acc[...] + b2[...]
        m = jnp.max(out2, axis=0, keepdims=True)
        sh = out2 - m
        lse = jnp.log(jnp.sum(jnp.exp(sh), axis=0, keepdims=True))
        o_ref[...] = sh - lse


def kernel(X, Z, adj_e, adj_v, T, w1, b1, p1, w2, b2, p2):
    N, E = T.shape
    NHID = w1.shape[1]
    NCLASS = w2.shape[1]
    Tt = T.T

    BJ1 = 256
    nj1 = N // BJ1
    d2 = pl.pallas_call(
        _node_kernel,
        grid=(nj1,),
        in_specs=[
            pl.BlockSpec((N, E), lambda j: (0, 0)),
            pl.BlockSpec((E, BJ1), lambda j: (0, j)),
            pl.BlockSpec((N, BJ1), lambda j: (0, j)),
            pl.BlockSpec((N, X.shape[1]), lambda j: (0, 0)),
            pl.BlockSpec((E, Z.shape[1]), lambda j: (0, 0)),
            pl.BlockSpec(w1.shape, lambda j: (0, 0)),
            pl.BlockSpec((1, NHID), lambda j: (0, 0)),
            pl.BlockSpec(p1.shape, lambda j: (0, 0)),
            pl.BlockSpec(p2.shape, lambda j: (0, 0)),
        ],
        out_specs=pl.BlockSpec((N, 1), lambda j: (0, 0)),
        out_shape=jax.ShapeDtypeStruct((N, 1), jnp.float32),
        scratch_shapes=[
            pltpu.VMEM((N, NHID), jnp.float32),
            pltpu.VMEM((N, NHID), jnp.float32),
        ],
        compiler_params=pltpu.CompilerParams(
            dimension_semantics=("arbitrary",)),
    )(T, Tt, adj_v, X, Z, w1, b1.reshape(1, NHID), p1, p2)

    BJ2 = 512
    nj2 = E // BJ2
    out = pl.pallas_call(
        _edge_kernel,
        grid=(nj2,),
        in_specs=[
            pl.BlockSpec((E, N), lambda j: (0, 0)),
            pl.BlockSpec((N, BJ2), lambda j: (0, j)),
            pl.BlockSpec((E, BJ2), lambda j: (0, j)),
            pl.BlockSpec((E, Z.shape[1]), lambda j: (0, 0)),
            pl.BlockSpec(w2.shape, lambda j: (0, 0)),
            pl.BlockSpec((1, NCLASS), lambda j: (0, 0)),
            pl.BlockSpec((N, 1), lambda j: (0, 0)),
        ],
        out_specs=pl.BlockSpec((E, NCLASS), lambda j: (0, 0)),
        out_shape=jax.ShapeDtypeStruct((E, NCLASS), jnp.float32),
        scratch_shapes=[pltpu.VMEM((E, NCLASS), jnp.float32)],
        compiler_params=pltpu.CompilerParams(
            dimension_semantics=("arbitrary",)),
    )(Tt, T, adj_e, Z, w2, b2.reshape(1, NCLASS), d2)
    return out


# speedup vs baseline: 1.0420x; 1.0420x over previous
import jax
import jax.numpy as jnp
from jax.experimental import pallas as pl
from jax.experimental.pallas import tpu as pltpu


def _node_kernel(Tf, Tt_t, adj_t, X, Z, w1, b1, p1, p2, d2_out, acc, hw):
    j = pl.program_id(0)
    nj = pl.num_programs(0)
    BJ = adj_t.shape[1]

    @pl.when(j == 0)
    def _init():
        hw[...] = jnp.dot(X[...], w1[...], preferred_element_type=jnp.float32)
        acc[...] = jnp.zeros_like(acc)

    d1 = jax.lax.dot_general(Z[...], p1[...], (((1,), (1,)), ((), ())),
                             preferred_element_type=jnp.float32)
    W = (Tt_t[...].astype(jnp.float32) * d1).astype(jnp.bfloat16)
    mult = jnp.dot(Tf[...], W, preferred_element_type=jnp.float32)

    ii = jax.lax.broadcasted_iota(jnp.int32, mult.shape, 0)
    jj = jax.lax.broadcasted_iota(jnp.int32, mult.shape, 1) + j * BJ
    A = jnp.where(ii == jj, adj_t[...], mult * adj_t[...])
    inv = 1.0 / (jnp.sum(A, axis=0) + 1e-10)
    V = hw[pl.ds(j * BJ, BJ), :] * inv[:, None]
    acc[...] += jnp.dot(A, V, preferred_element_type=jnp.float32)

    @pl.when(j == nj - 1)
    def _fin():
        Xv = jnp.maximum(acc[...] + b1[...], 0.0)
        d2_out[...] = jax.lax.dot_general(
            Xv, p2[...], (((1,), (1,)), ((), ())),
            preferred_element_type=jnp.float32)


def _edge_kernel(Tt, Tcol_t, adj_t, Z, w2, b2, d2, o_ref, acc):
    j = pl.program_id(0)
    nj = pl.num_programs(0)
    BJ = adj_t.shape[1]

    @pl.when(j == 0)
    def _init():
        acc[...] = jnp.zeros_like(acc)

    W = (Tcol_t[...].astype(jnp.float32) * d2[...]).astype(jnp.bfloat16)
    mult = jnp.dot(Tt[...], W, preferred_element_type=jnp.float32)

    ii = jax.lax.broadcasted_iota(jnp.int32, mult.shape, 0)
    jj = jax.lax.broadcasted_iota(jnp.int32, mult.shape, 1) + j * BJ
    A = jnp.where(ii == jj, adj_t[...], mult * adj_t[...])
    inv = 1.0 / (jnp.sum(A, axis=0) + 1e-10)

    Ze_t = jnp.maximum(Z[pl.ds(j * BJ, BJ), :], 0.0)
    HW2_t = jnp.dot(Ze_t, w2[...], preferred_element_type=jnp.float32)
    V = HW2_t * inv[:, None]
    acc[...] += jnp.dot(A, V, preferred_element_type=jnp.float32)

    @pl.when(j == nj - 1)
    def _fin():
        out2 = acc[...] + b2[...]
        m = jnp.max(out2, axis=0, keepdims=True)
        sh = out2 - m
        lse = jnp.log(jnp.sum(jnp.exp(sh), axis=0, keepdims=True))
        o_ref[...] = sh - lse


def kernel(X, Z, adj_e, adj_v, T, w1, b1, p1, w2, b2, p2):
    N, E = T.shape
    NHID = w1.shape[1]
    NCLASS = w2.shape[1]
    T_bf = T.astype(jnp.bfloat16)
    Tt_bf = T_bf.T

    BJ1 = 256
    nj1 = N // BJ1
    d2 = pl.pallas_call(
        _node_kernel,
        grid=(nj1,),
        in_specs=[
            pl.BlockSpec((N, E), lambda j: (0, 0)),
            pl.BlockSpec((E, BJ1), lambda j: (0, j)),
            pl.BlockSpec((N, BJ1), lambda j: (0, j)),
            pl.BlockSpec((N, X.shape[1]), lambda j: (0, 0)),
            pl.BlockSpec((E, Z.shape[1]), lambda j: (0, 0)),
            pl.BlockSpec(w1.shape, lambda j: (0, 0)),
            pl.BlockSpec((1, NHID), lambda j: (0, 0)),
            pl.BlockSpec(p1.shape, lambda j: (0, 0)),
            pl.BlockSpec(p2.shape, lambda j: (0, 0)),
        ],
        out_specs=pl.BlockSpec((N, 1), lambda j: (0, 0)),
        out_shape=jax.ShapeDtypeStruct((N, 1), jnp.float32),
        scratch_shapes=[
            pltpu.VMEM((N, NHID), jnp.float32),
            pltpu.VMEM((N, NHID), jnp.float32),
        ],
        compiler_params=pltpu.CompilerParams(
            dimension_semantics=("arbitrary",)),
    )(T_bf, Tt_bf, adj_v, X, Z, w1, b1.reshape(1, NHID), p1, p2)

    BJ2 = 512
    nj2 = E // BJ2
    out = pl.pallas_call(
        _edge_kernel,
        grid=(nj2,),
        in_specs=[
            pl.BlockSpec((E, N), lambda j: (0, 0)),
            pl.BlockSpec((N, BJ2), lambda j: (0, j)),
            pl.BlockSpec((E, BJ2), lambda j: (0, j)),
            pl.BlockSpec((E, Z.shape[1]), lambda j: (0, 0)),
            pl.BlockSpec(w2.shape, lambda j: (0, 0)),
            pl.BlockSpec((1, NCLASS), lambda j: (0, 0)),
            pl.BlockSpec((N, 1), lambda j: (0, 0)),
        ],
        out_specs=pl.BlockSpec((E, NCLASS), lambda j: (0, 0)),
        out_shape=jax.ShapeDtypeStruct((E, NCLASS), jnp.float32),
        scratch_shapes=[pltpu.VMEM((E, NCLASS), jnp.float32)],
        compiler_params=pltpu.CompilerParams(
            dimension_semantics=("arbitrary",)),
    )(Tt_bf, T_bf, adj_e, Z, w2, b2.reshape(1, NCLASS), d2)
    return out
